# baseline (speedup 1.0000x reference)
import functools

import jax
import jax.numpy as jnp
from jax.experimental import pallas as pl
from jax.experimental.pallas import tpu as pltpu

B, N, D = 256, 32, 128
MB = 4
NS = B // MB
G = 2
R = N // G
TRIPS = MB * G
CR = MB * N * N // TRIPS
HR = CR // N


def _fwl_kernel(x_ref, w10_ref, b10_ref, w11_ref, b11_ref,
                w20_ref, b20_ref, w21_ref, b21_ref, out_ref,
                h1_ref, h2_ref, x1_ref, x2_ref):
    s = pl.program_id(0)
    slot_h_w = jax.lax.rem(s, 2)
    slot_h_r = 1 - slot_h_w
    slot_x_w = 1 - slot_h_w
    slot_x_r = slot_h_w

    def trip(t, carry):
        m = t // G
        half = jax.lax.rem(t, G)
        i0 = half * R

        accs = [None] * R
        for k in range(N):
            b_k = x2_ref[slot_x_r, m, k]
            for ri in range(R):
                t_ = x1_ref[slot_x_r, m, i0 + ri, k:k + 1, :] * b_k
                accs[ri] = t_ if accs[ri] is None else accs[ri] + t_
        for ri in range(R):
            out_ref[m, i0 + ri] = accs[ri]
        xc = x_ref[m, pl.ds(i0, HR)].reshape(CR, D)
        h1 = jnp.maximum(
            jnp.dot(xc, w10_ref[...], preferred_element_type=jnp.float32)
            + b10_ref[...], 0.0)
        h1_ref[slot_h_w, pl.ds(t * CR, CR), :] = h1
        h2 = jnp.maximum(
            jnp.dot(xc, w20_ref[...], preferred_element_type=jnp.float32)
            + b20_ref[...], 0.0)
        h2_ref[slot_h_w, pl.ds(t * CR, CR), :] = h2

        hp1 = h1_ref[slot_h_r, pl.ds(t * CR, CR), :]
        x1_ref[slot_x_w, m, pl.ds(i0, HR)] = jnp.maximum(
            jnp.dot(hp1, w11_ref[...], preferred_element_type=jnp.float32)
            + b11_ref[...], 0.0).reshape(HR, N, D)
        hp2 = h2_ref[slot_h_r, pl.ds(t * CR, CR), :]
        x2_ref[slot_x_w, m, pl.ds(i0, HR)] = jnp.maximum(
            jnp.dot(hp2, w21_ref[...], preferred_element_type=jnp.float32)
            + b21_ref[...], 0.0).reshape(HR, N, D)

        return carry

    jax.lax.fori_loop(0, TRIPS, trip, 0, unroll=False)


@functools.partial(jax.jit, static_argnames=())
def kernel(x_data, x_mask, W1_0, b1_0, W1_1, b1_1, W2_0, b2_0, W2_1, b2_1):
    del x_mask
    w_spec = pl.BlockSpec((D, D), lambda s: (0, 0))
    b_spec = pl.BlockSpec((1, D), lambda s: (0, 0))
    return pl.pallas_call(
        _fwl_kernel,
        grid=(NS + 2,),
        in_specs=[
            pl.BlockSpec((MB, N, N, D),
                         lambda s: (jnp.minimum(s, NS - 1), 0, 0, 0)),
            w_spec, b_spec, w_spec, b_spec,
            w_spec, b_spec, w_spec, b_spec,
        ],
        out_specs=pl.BlockSpec((MB, N, N, D),
                               lambda s: (jnp.maximum(s - 2, 0), 0, 0, 0)),
        out_shape=jax.ShapeDtypeStruct((B, N, N, D), jnp.float32),
        scratch_shapes=[
            pltpu.VMEM((2, MB * N * N, D), jnp.float32),
            pltpu.VMEM((2, MB * N * N, D), jnp.float32),
            pltpu.VMEM((2, MB, N, N, D), jnp.float32),
            pltpu.VMEM((2, MB, N, N, D), jnp.float32),
        ],
    )(x_data, W1_0, b1_0.reshape(1, D), W1_1, b1_1.reshape(1, D),
      W2_0, b2_0.reshape(1, D), W2_1, b2_1.reshape(1, D))


# speedup vs baseline: 1.3358x; 1.3358x over previous
import functools

import jax
import jax.numpy as jnp
from jax.experimental import pallas as pl
from jax.experimental.pallas import tpu as pltpu

B, N, D = 256, 32, 128
MB = 8


def _fwl_kernel(x_ref, w10_ref, b10_ref, w11_ref, b11_ref,
                w20_ref, b20_ref, w21_ref, b21_ref, out_ref,
                x1_ref, x2_ref):
    x = x_ref[...].reshape(MB * N * N, D)

    h = jnp.maximum(jnp.dot(x, w10_ref[...], preferred_element_type=jnp.float32)
                    + b10_ref[...], 0.0)
    x1_ref[...] = jnp.maximum(
        jnp.dot(h, w11_ref[...], preferred_element_type=jnp.float32)
        + b11_ref[...], 0.0).reshape(MB, N, N, D)
    h = jnp.maximum(jnp.dot(x, w20_ref[...], preferred_element_type=jnp.float32)
                    + b20_ref[...], 0.0)
    x2_ref[...] = jnp.maximum(
        jnp.dot(h, w21_ref[...], preferred_element_type=jnp.float32)
        + b21_ref[...], 0.0).reshape(MB, N, N, D)

    R = 16
    def rows(m, carry):
        for g in range(N // R):
            i = g * R
            accs = [None] * R
            for k in range(N):
                b_k = x2_ref[m, k]
                for ri in range(R):
                    t = x1_ref[m, i + ri, k:k + 1, :] * b_k
                    accs[ri] = t if accs[ri] is None else accs[ri] + t
            for ri in range(R):
                out_ref[m, i + ri] = accs[ri]
        return carry

    jax.lax.fori_loop(0, MB, rows, 0, unroll=False)


@functools.partial(jax.jit, static_argnames=())
def kernel(x_data, x_mask, W1_0, b1_0, W1_1, b1_1, W2_0, b2_0, W2_1, b2_1):
    del x_mask
    w_spec = pl.BlockSpec((D, D), lambda b: (0, 0))
    b_spec = pl.BlockSpec((1, D), lambda b: (0, 0))
    return pl.pallas_call(
        _fwl_kernel,
        grid=(B // MB,),
        in_specs=[
            pl.BlockSpec((MB, N, N, D), lambda b: (b, 0, 0, 0)),
            w_spec, b_spec, w_spec, b_spec,
            w_spec, b_spec, w_spec, b_spec,
        ],
        out_specs=pl.BlockSpec((MB, N, N, D), lambda b: (b, 0, 0, 0)),
        out_shape=jax.ShapeDtypeStruct((B, N, N, D), jnp.float32),
        scratch_shapes=[
            pltpu.VMEM((MB, N, N, D), jnp.float32),
            pltpu.VMEM((MB, N, N, D), jnp.float32),
        ],
    )(x_data, W1_0, b1_0.reshape(1, D), W1_1, b1_1.reshape(1, D),
      W2_0, b2_0.reshape(1, D), W2_1, b2_1.reshape(1, D))


# speedup vs baseline: 1.3494x; 1.0101x over previous
import functools

import jax
import jax.numpy as jnp
from jax.experimental import pallas as pl
from jax.experimental.pallas import tpu as pltpu

B, N, D = 256, 32, 128
MB = 16


def _fwl_kernel(x_ref, w10_ref, b10_ref, w11_ref, b11_ref,
                w20_ref, b20_ref, w21_ref, b21_ref, out_ref,
                x1_ref, x2_ref):
    x = x_ref[...].reshape(MB * N * N, D)

    h = jnp.maximum(jnp.dot(x, w10_ref[...], preferred_element_type=jnp.float32)
                    + b10_ref[...], 0.0)
    x1_ref[...] = jnp.maximum(
        jnp.dot(h, w11_ref[...], preferred_element_type=jnp.float32)
        + b11_ref[...], 0.0).reshape(MB, N, N, D)
    h = jnp.maximum(jnp.dot(x, w20_ref[...], preferred_element_type=jnp.float32)
                    + b20_ref[...], 0.0)
    x2_ref[...] = jnp.maximum(
        jnp.dot(h, w21_ref[...], preferred_element_type=jnp.float32)
        + b21_ref[...], 0.0).reshape(MB, N, N, D)

    R = 16
    def rows(m, carry):
        for g in range(N // R):
            i = g * R
            accs = [None] * R
            for k in range(N):
                b_k = x2_ref[m, k]
                for ri in range(R):
                    t = x1_ref[m, i + ri, k:k + 1, :] * b_k
                    accs[ri] = t if accs[ri] is None else accs[ri] + t
            for ri in range(R):
                out_ref[m, i + ri] = accs[ri]
        return carry

    jax.lax.fori_loop(0, MB, rows, 0, unroll=False)


@functools.partial(jax.jit, static_argnames=())
def kernel(x_data, x_mask, W1_0, b1_0, W1_1, b1_1, W2_0, b2_0, W2_1, b2_1):
    del x_mask
    w_spec = pl.BlockSpec((D, D), lambda b: (0, 0))
    b_spec = pl.BlockSpec((1, D), lambda b: (0, 0))
    return pl.pallas_call(
        _fwl_kernel,
        grid=(B // MB,),
        in_specs=[
            pl.BlockSpec((MB, N, N, D), lambda b: (b, 0, 0, 0)),
            w_spec, b_spec, w_spec, b_spec,
            w_spec, b_spec, w_spec, b_spec,
        ],
        out_specs=pl.BlockSpec((MB, N, N, D), lambda b: (b, 0, 0, 0)),
        out_shape=jax.ShapeDtypeStruct((B, N, N, D), jnp.float32),
        scratch_shapes=[
            pltpu.VMEM((MB, N, N, D), jnp.float32),
            pltpu.VMEM((MB, N, N, D), jnp.float32),
        ],
    )(x_data, W1_0, b1_0.reshape(1, D), W1_1, b1_1.reshape(1, D),
      W2_0, b2_0.reshape(1, D), W2_1, b2_1.reshape(1, D))


# speedup vs baseline: 1.3616x; 1.0091x over previous
import functools

import jax
import jax.numpy as jnp
from jax.experimental import pallas as pl
from jax.experimental.pallas import tpu as pltpu

B, N, D = 256, 32, 128
MB = 16


def _fwl_kernel(x_ref, w10_ref, b10_ref, w11_ref, b11_ref,
                w20_ref, b20_ref, w21_ref, b21_ref, out_ref,
                x1_ref, x2_ref):
    x = x_ref[...].reshape(MB * N * N, D)

    h = jnp.maximum(jnp.dot(x, w10_ref[...], preferred_element_type=jnp.float32)
                    + b10_ref[...], 0.0)
    x1_ref[...] = jnp.maximum(
        jnp.dot(h, w11_ref[...], preferred_element_type=jnp.float32)
        + b11_ref[...], 0.0).reshape(MB, N, N, D)
    h = jnp.maximum(jnp.dot(x, w20_ref[...], preferred_element_type=jnp.float32)
                    + b20_ref[...], 0.0)
    x2_ref[...] = jnp.maximum(
        jnp.dot(h, w21_ref[...], preferred_element_type=jnp.float32)
        + b21_ref[...], 0.0).reshape(MB, N, N, D)

    R = 16
    def rows(m, carry):
        for g in range(N // R):
            i = g * R
            accs = [None] * R
            for k in range(N):
                b_k = x2_ref[m, k]
                for ri in range(R):
                    t = x1_ref[m, i + ri, k:k + 1, :] * b_k
                    accs[ri] = t if accs[ri] is None else accs[ri] + t
            for ri in range(R):
                out_ref[m, i + ri] = accs[ri]
        return carry

    jax.lax.fori_loop(0, MB, rows, 0, unroll=2)


@functools.partial(jax.jit, static_argnames=())
def kernel(x_data, x_mask, W1_0, b1_0, W1_1, b1_1, W2_0, b2_0, W2_1, b2_1):
    del x_mask
    w_spec = pl.BlockSpec((D, D), lambda b: (0, 0))
    b_spec = pl.BlockSpec((1, D), lambda b: (0, 0))
    return pl.pallas_call(
        _fwl_kernel,
        grid=(B // MB,),
        in_specs=[
            pl.BlockSpec((MB, N, N, D), lambda b: (b, 0, 0, 0)),
            w_spec, b_spec, w_spec, b_spec,
            w_spec, b_spec, w_spec, b_spec,
        ],
        out_specs=pl.BlockSpec((MB, N, N, D), lambda b: (b, 0, 0, 0)),
        out_shape=jax.ShapeDtypeStruct((B, N, N, D), jnp.float32),
        scratch_shapes=[
            pltpu.VMEM((MB, N, N, D), jnp.float32),
            pltpu.VMEM((MB, N, N, D), jnp.float32),
        ],
    )(x_data, W1_0, b1_0.reshape(1, D), W1_1, b1_1.reshape(1, D),
      W2_0, b2_0.reshape(1, D), W2_1, b2_1.reshape(1, D))


# speedup vs baseline: 1.3677x; 1.0045x over previous
import functools

import jax
import jax.numpy as jnp
from jax.experimental import pallas as pl
from jax.experimental.pallas import tpu as pltpu

B, N, D = 256, 32, 128
MB = 16


def _fwl_kernel(x_ref, w10_ref, b10_ref, w11_ref, b11_ref,
                w20_ref, b20_ref, w21_ref, b21_ref, out_ref,
                x1_ref, x2_ref):
    x = x_ref[...].reshape(MB * N * N, D)

    h = jnp.maximum(jnp.dot(x, w10_ref[...], preferred_element_type=jnp.float32)
                    + b10_ref[...], 0.0)
    x1_ref[...] = jnp.maximum(
        jnp.dot(h, w11_ref[...], preferred_element_type=jnp.float32)
        + b11_ref[...], 0.0).reshape(MB, N, N, D)
    h = jnp.maximum(jnp.dot(x, w20_ref[...], preferred_element_type=jnp.float32)
                    + b20_ref[...], 0.0)
    x2_ref[...] = jnp.maximum(
        jnp.dot(h, w21_ref[...], preferred_element_type=jnp.float32)
        + b21_ref[...], 0.0).reshape(MB, N, N, D)

    R = 16
    def rows(m, carry):
        for g in range(N // R):
            i = g * R
            accs = [None] * R
            for k in range(N):
                b_k = x2_ref[m, k]
                for ri in range(R):
                    t = x1_ref[m, i + ri, k:k + 1, :] * b_k
                    accs[ri] = t if accs[ri] is None else accs[ri] + t
            for ri in range(R):
                out_ref[m, i + ri] = accs[ri]
        return carry

    jax.lax.fori_loop(0, MB, rows, 0, unroll=4)


@functools.partial(jax.jit, static_argnames=())
def kernel(x_data, x_mask, W1_0, b1_0, W1_1, b1_1, W2_0, b2_0, W2_1, b2_1):
    del x_mask
    w_spec = pl.BlockSpec((D, D), lambda b: (0, 0))
    b_spec = pl.BlockSpec((1, D), lambda b: (0, 0))
    return pl.pallas_call(
        _fwl_kernel,
        grid=(B // MB,),
        in_specs=[
            pl.BlockSpec((MB, N, N, D), lambda b: (b, 0, 0, 0)),
            w_spec, b_spec, w_spec, b_spec,
            w_spec, b_spec, w_spec, b_spec,
        ],
        out_specs=pl.BlockSpec((MB, N, N, D), lambda b: (b, 0, 0, 0)),
        out_shape=jax.ShapeDtypeStruct((B, N, N, D), jnp.float32),
        scratch_shapes=[
            pltpu.VMEM((MB, N, N, D), jnp.float32),
            pltpu.VMEM((MB, N, N, D), jnp.float32),
        ],
    )(x_data, W1_0, b1_0.reshape(1, D), W1_1, b1_1.reshape(1, D),
      W2_0, b2_0.reshape(1, D), W2_1, b2_1.reshape(1, D))
